# baseline (speedup 1.0000x reference)
import jax
import jax.numpy as jnp
from jax import lax
from jax.experimental import pallas as pl
from jax.experimental.pallas import tpu as pltpu

_P = 30
_K = 20
_L = 7
_KS = 5
_F = 10
_G = 128
_TPB = 14
_KNOTS = (0.0, 0.5, 1.0, 1.5, 2.0)
_INV_H = 2.0
_CHUNKS = ((0, 8), (8, 8), (16, 8), (24, 6))


def _selection_pairs(n, out_idx):
    pairs = []
    p = 1
    while p < n:
        k = p
        while k >= 1:
            for j in range(k % p, n - k, 2 * k):
                for i in range(min(k, n - j - k)):
                    if (i + j) // (p * 2) == (i + j + k) // (p * 2):
                        pairs.append((i + j, i + j + k))
            k //= 2
        p *= 2
    keep = {out_idx}
    kept = []
    for a, b in reversed(pairs):
        if a in keep or b in keep:
            kept.append((a, b))
            keep.add(a)
            keep.add(b)
    kept.reverse()
    return tuple(kept)


_SORT_PAIRS = _selection_pairs(32, _K - 1)


def _sub_tile(p, D, W_ref):
    px, py, pz = p[0], p[1], p[2]

    pxb = jnp.broadcast_to(px[:, None, :], (_P, 8, _G))
    pyb = jnp.broadcast_to(py[:, None, :], (_P, 8, _G))
    pzb = jnp.broadcast_to(pz[:, None, :], (_P, 8, _G))

    S = [[[] for _ in range(_L)] for _ in range(_KS)]
    for i0, isz in _CHUNKS:
        pxc = px[i0:i0 + isz]
        pyc = py[i0:i0 + isz]
        pzc = pz[i0:i0 + isz]

        dxs, dys, dzs, d2s, d2rs = [], [], [], [], []
        iio = lax.broadcasted_iota(jnp.int32, (isz, 1), 0)
        for j in range(_P):
            dxj = pxb[j, :isz] - pxc
            dyj = pyb[j, :isz] - pyc
            dzj = pzb[j, :isz] - pzc
            d2j = dxj * dxj + dyj * dyj + dzj * dzj
            dxs.append(dxj)
            dys.append(dyj)
            dzs.append(dzj)
            d2s.append(d2j)
            if i0 <= j < i0 + isz:
                d2rs.append(d2j + jnp.where(iio == j - i0, 1e9, 0.0))
            else:
                d2rs.append(d2j)

        pad = jnp.full((isz, _G), jnp.inf, jnp.float32)
        vals = list(d2rs) + [pad, pad]
        for a, b in _SORT_PAIRS:
            lo = jnp.minimum(vals[a], vals[b])
            hi = jnp.maximum(vals[a], vals[b])
            vals[a], vals[b] = lo, hi
        T0 = vals[_K - 1]

        need = jnp.zeros((isz, _G), jnp.float32)
        for m in range(_K):
            need = need + (vals[m] == T0).astype(jnp.float32)

        acc = [[jnp.zeros((isz, _G), jnp.float32) for _ in range(_L)]
               for _ in range(_KS)]
        cnt = jnp.zeros((isz, _G), jnp.float32)
        for j in range(_P):
            d2rj = d2rs[j]
            eqj = d2rj == T0
            cnt = cnt + eqj.astype(jnp.float32)
            selj = (d2rj < T0) | (eqj & (cnt <= need))
            rj = jnp.sqrt(d2s[j])
            wj = jnp.where(selj, 1.0 / (rj + 1e-8), 0.0)
            mb = [wj * jnp.maximum(0.0, 1.0 - jnp.abs(rj - _KNOTS[k]) * _INV_H)
                  for k in range(_KS)]
            ft = [jnp.maximum(dxs[j] * D[0][l] + dys[j] * D[1][l]
                              + dzs[j] * D[2][l], 0.0)
                  for l in range(_L)]
            for k in range(_KS):
                for l in range(_L):
                    acc[k][l] = acc[k][l] + mb[k] * ft[l]
        for k in range(_KS):
            for l in range(_L):
                S[k][l].append(acc[k][l])

    Sf = [[jnp.concatenate(S[k][l], axis=0) for l in range(_L)] for k in range(_KS)]

    accs = [jnp.zeros((_P, _G), jnp.float32) for _ in range(_F)]
    for k in range(_KS):
        for l in range(_L):
            s = Sf[k][l]
            for f in range(_F):
                accs[f] = accs[f] + s * W_ref[k * _L + l, f]
    ys_rows = [
        jnp.mean(jax.nn.sigmoid(accs[f] * (1.0 / _K)), axis=0, keepdims=True)
        for f in range(_F)
    ]
    return jnp.concatenate(ys_rows, axis=0)


def _body(pos_ref, D_ref, W_ref, W1T_ref, b1_ref, W2T_ref, b2_ref, out_ref):
    D = [[D_ref[c, l] for l in range(_L)] for c in range(3)]
    for s in range(_TPB):
        ys = _sub_tile(pos_ref[0, s], D, W_ref)
        z1 = jnp.dot(W1T_ref[...], ys, preferred_element_type=jnp.float32) + b1_ref[...]
        y1 = jnp.where(z1 > 0, z1, jnp.exp(z1) - 1.0)
        z2 = jnp.dot(W2T_ref[...], y1, preferred_element_type=jnp.float32) + b2_ref[...]
        m = jnp.max(z2, axis=0, keepdims=True)
        lse = jnp.log(jnp.sum(jnp.exp(z2 - m), axis=0, keepdims=True))
        out_ref[0, s] = z2 - m - lse


@jax.jit
def kernel(pos, edge_index, batch, D, W, W1, b1, W2, b2):
    del edge_index, batch
    n = pos.shape[0]
    B = n // _P
    nt = (B + _G - 1) // _G
    T = (nt + _TPB - 1) // _TPB
    npad = T * _TPB * _G * _P - n
    posp = jnp.pad(pos, ((0, npad), (0, 0)))
    pos_t = posp.reshape(T * _TPB, _G, _P, 3).transpose(0, 3, 2, 1)
    pos_t = pos_t.reshape(T, _TPB, 3, _P, _G)

    nc = W2.shape[1]
    out = pl.pallas_call(
        _body,
        grid=(T,),
        in_specs=[
            pl.BlockSpec((1, _TPB, 3, _P, _G), lambda t: (t, 0, 0, 0, 0)),
            pl.BlockSpec(memory_space=pltpu.SMEM),
            pl.BlockSpec(memory_space=pltpu.SMEM),
            pl.BlockSpec((256, _F), lambda t: (0, 0)),
            pl.BlockSpec((256, 1), lambda t: (0, 0)),
            pl.BlockSpec((nc, 256), lambda t: (0, 0)),
            pl.BlockSpec((nc, 1), lambda t: (0, 0)),
        ],
        out_specs=pl.BlockSpec((1, _TPB, nc, _G), lambda t: (t, 0, 0, 0)),
        out_shape=jax.ShapeDtypeStruct((T, _TPB, nc, _G), jnp.float32),
        compiler_params=pltpu.CompilerParams(
            dimension_semantics=("arbitrary",),
        ),
    )(pos_t, D, W, W1.T, b1[:, None], W2.T, b2[:, None])
    return out.transpose(0, 1, 3, 2).reshape(T * _TPB * _G, nc)[:B]


# speedup vs baseline: 1.0449x; 1.0449x over previous
import jax
import jax.numpy as jnp
from jax import lax
from jax.experimental import pallas as pl
from jax.experimental.pallas import tpu as pltpu

_P = 30
_K = 20
_L = 7
_KS = 5
_F = 10
_G = 128
_TPB = 7
_KNOTS = (0.0, 0.5, 1.0, 1.5, 2.0)
_INV_H = 2.0
_CHUNKS = ((0, 8), (8, 8), (16, 8), (24, 6))


def _selection_pairs(n, out_idx):
    pairs = []
    p = 1
    while p < n:
        k = p
        while k >= 1:
            for j in range(k % p, n - k, 2 * k):
                for i in range(min(k, n - j - k)):
                    if (i + j) // (p * 2) == (i + j + k) // (p * 2):
                        pairs.append((i + j, i + j + k))
            k //= 2
        p *= 2
    keep = {out_idx}
    kept = []
    for a, b in reversed(pairs):
        if a in keep or b in keep:
            kept.append((a, b))
            keep.add(a)
            keep.add(b)
    kept.reverse()
    return tuple(kept)


_SORT_PAIRS = _selection_pairs(32, _K - 1)


def _sub_tile(p, D, W_ref):
    px, py, pz = p[0], p[1], p[2]

    pxb = jnp.broadcast_to(px[:, None, :], (_P, 8, _G))
    pyb = jnp.broadcast_to(py[:, None, :], (_P, 8, _G))
    pzb = jnp.broadcast_to(pz[:, None, :], (_P, 8, _G))

    S = [[[] for _ in range(_L)] for _ in range(_KS)]
    for i0, isz in _CHUNKS:
        pxc = px[i0:i0 + isz]
        pyc = py[i0:i0 + isz]
        pzc = pz[i0:i0 + isz]

        dxs, dys, dzs, d2s, d2rs = [], [], [], [], []
        iio = lax.broadcasted_iota(jnp.int32, (isz, 1), 0)
        for j in range(_P):
            dxj = pxb[j, :isz] - pxc
            dyj = pyb[j, :isz] - pyc
            dzj = pzb[j, :isz] - pzc
            d2j = dxj * dxj + dyj * dyj + dzj * dzj
            dxs.append(dxj)
            dys.append(dyj)
            dzs.append(dzj)
            d2s.append(d2j)
            if i0 <= j < i0 + isz:
                d2rs.append(d2j + jnp.where(iio == j - i0, 1e9, 0.0))
            else:
                d2rs.append(d2j)

        pad = jnp.full((isz, _G), jnp.inf, jnp.float32)
        vals = list(d2rs) + [pad, pad]
        for a, b in _SORT_PAIRS:
            lo = jnp.minimum(vals[a], vals[b])
            hi = jnp.maximum(vals[a], vals[b])
            vals[a], vals[b] = lo, hi
        T0 = vals[_K - 1]

        need = jnp.zeros((isz, _G), jnp.float32)
        for m in range(_K):
            need = need + (vals[m] == T0).astype(jnp.float32)

        acc = [[jnp.zeros((isz, _G), jnp.float32) for _ in range(_L)]
               for _ in range(_KS)]
        cnt = jnp.zeros((isz, _G), jnp.float32)
        for j in range(_P):
            d2rj = d2rs[j]
            eqj = d2rj == T0
            cnt = cnt + eqj.astype(jnp.float32)
            selj = (d2rj < T0) | (eqj & (cnt <= need))
            rj = jnp.sqrt(d2s[j])
            wj = jnp.where(selj, 1.0 / (rj + 1e-8), 0.0)
            mb = [wj * jnp.maximum(0.0, 1.0 - jnp.abs(rj - _KNOTS[k]) * _INV_H)
                  for k in range(_KS)]
            ft = [jnp.maximum(dxs[j] * D[0][l] + dys[j] * D[1][l]
                              + dzs[j] * D[2][l], 0.0)
                  for l in range(_L)]
            for k in range(_KS):
                for l in range(_L):
                    acc[k][l] = acc[k][l] + mb[k] * ft[l]
        for k in range(_KS):
            for l in range(_L):
                S[k][l].append(acc[k][l])

    Sf = [[jnp.concatenate(S[k][l], axis=0) for l in range(_L)] for k in range(_KS)]

    accs = [jnp.zeros((_P, _G), jnp.float32) for _ in range(_F)]
    for k in range(_KS):
        for l in range(_L):
            s = Sf[k][l]
            for f in range(_F):
                accs[f] = accs[f] + s * W_ref[k * _L + l, f]
    ys_rows = [
        jnp.mean(jax.nn.sigmoid(accs[f] * (1.0 / _K)), axis=0, keepdims=True)
        for f in range(_F)
    ]
    return jnp.concatenate(ys_rows, axis=0)


def _body(pos_ref, D_ref, W_ref, W1T_ref, b1_ref, W2T_ref, b2_ref, out_ref):
    D = [[D_ref[c, l] for l in range(_L)] for c in range(3)]
    for s in range(_TPB):
        ys = _sub_tile(pos_ref[0, s], D, W_ref)
        z1 = jnp.dot(W1T_ref[...], ys, preferred_element_type=jnp.float32) + b1_ref[...]
        y1 = jnp.where(z1 > 0, z1, jnp.exp(z1) - 1.0)
        z2 = jnp.dot(W2T_ref[...], y1, preferred_element_type=jnp.float32) + b2_ref[...]
        m = jnp.max(z2, axis=0, keepdims=True)
        lse = jnp.log(jnp.sum(jnp.exp(z2 - m), axis=0, keepdims=True))
        out_ref[0, s] = z2 - m - lse


@jax.jit
def kernel(pos, edge_index, batch, D, W, W1, b1, W2, b2):
    del edge_index, batch
    n = pos.shape[0]
    B = n // _P
    nt = (B + _G - 1) // _G
    T = (nt + _TPB - 1) // _TPB
    npad = T * _TPB * _G * _P - n
    posp = jnp.pad(pos, ((0, npad), (0, 0)))
    pos_t = posp.reshape(T * _TPB, _G, _P, 3).transpose(0, 3, 2, 1)
    pos_t = pos_t.reshape(T, _TPB, 3, _P, _G)

    nc = W2.shape[1]
    out = pl.pallas_call(
        _body,
        grid=(T,),
        in_specs=[
            pl.BlockSpec((1, _TPB, 3, _P, _G), lambda t: (t, 0, 0, 0, 0)),
            pl.BlockSpec(memory_space=pltpu.SMEM),
            pl.BlockSpec(memory_space=pltpu.SMEM),
            pl.BlockSpec((256, _F), lambda t: (0, 0)),
            pl.BlockSpec((256, 1), lambda t: (0, 0)),
            pl.BlockSpec((nc, 256), lambda t: (0, 0)),
            pl.BlockSpec((nc, 1), lambda t: (0, 0)),
        ],
        out_specs=pl.BlockSpec((1, _TPB, nc, _G), lambda t: (t, 0, 0, 0)),
        out_shape=jax.ShapeDtypeStruct((T, _TPB, nc, _G), jnp.float32),
        compiler_params=pltpu.CompilerParams(
            dimension_semantics=("arbitrary",),
        ),
    )(pos_t, D, W, W1.T, b1[:, None], W2.T, b2[:, None])
    return out.transpose(0, 1, 3, 2).reshape(T * _TPB * _G, nc)[:B]
